# baseline (speedup 1.0000x reference)
import functools

import jax
import jax.numpy as jnp
from jax import lax
from jax.experimental import pallas as pl
from jax.experimental.pallas import tpu as pltpu

LANE = 128


def _stats_kernel(x_ref, g_ref, b_ref, mx_ref, mn_ref, accmx_ref, accmn_ref):
    n = pl.program_id(1)
    i = pl.program_id(2)

    @pl.when((n == 0) & (i == 0))
    def _():
        accmx_ref[...] = jnp.full(accmx_ref.shape, -jnp.inf, accmx_ref.dtype)
        accmn_ref[...] = jnp.full(accmn_ref.shape, jnp.inf, accmn_ref.dtype)

    x = x_ref[0]
    x2 = (x * x).astype(jnp.bfloat16)
    xx = jnp.dot(g_ref[...], x2, preferred_element_type=jnp.float32)
    accmx_ref[...] = jnp.maximum(accmx_ref[...], xx)
    accmn_ref[...] = jnp.minimum(accmn_ref[...], xx)

    @pl.when((n == pl.num_programs(1) - 1) & (i == pl.num_programs(2) - 1))
    def _():
        b = b_ref[...]
        mx_ref[0] = jnp.max(accmx_ref[...], axis=1, keepdims=True) + b
        mn_ref[0] = jnp.min(accmn_ref[...], axis=1, keepdims=True) + b


def _quant_kernel(x_ref, g_ref, b_ref, s1_ref, mxp_ref, mnp_ref, o_ref,
                  *, qn, qp, inverse):
    mx = jnp.max(mxp_ref[...], axis=0)
    mn = jnp.min(mnp_ref[...], axis=0)
    qscl = (mx - mn) * (0.9 / (qp - qn))
    qoff = mn * 0.9 - qn * qscl
    inv_qscl = 1.0 / qscl

    x = x_ref[0]
    x2 = (x * x).astype(jnp.bfloat16)
    xx = jnp.dot(g_ref[...], x2,
                 preferred_element_type=jnp.float32) + b_ref[...]

    x_hat = jnp.clip(jnp.round((xx - qoff) * inv_qscl), qn, qp)
    xq = x_hat * qscl + qoff

    if inverse:
        norm = jnp.sqrt(xq)
    else:
        norm = lax.rsqrt(xq)

    o_ref[0] = (s1_ref[...] * x) * norm


def _gdn_forward(x_nchw, gamma, beta, s1, *, num=256, inverse=False):
    N, C, H, W = x_nchw.shape
    HW = H * W
    qn, qp = 0.0, float(num - 1)

    T = LANE
    for cand_t in (2048, 1024, 512, 256, LANE):
        if HW % cand_t == 0:
            T = cand_t
            break
    J = 2 if N % 2 == 0 else 1

    x3 = x_nchw.reshape(N, C, HW)
    g_bf = gamma.astype(jnp.bfloat16)
    b_col = beta.astype(jnp.float32).reshape(C, 1)
    s1_col = s1.astype(jnp.float32).reshape(C, 1)

    vmem_limit = 48 * 1024 * 1024

    mxp, mnp = pl.pallas_call(
        _stats_kernel,
        out_shape=(
            jax.ShapeDtypeStruct((J, C, 1), jnp.float32),
            jax.ShapeDtypeStruct((J, C, 1), jnp.float32),
        ),
        grid_spec=pltpu.PrefetchScalarGridSpec(
            num_scalar_prefetch=0,
            grid=(J, N // J, HW // T),
            in_specs=[
                pl.BlockSpec((1, C, T), lambda j, n, i: (j * (N // J) + n, 0, i)),
                pl.BlockSpec((C, C), lambda j, n, i: (0, 0)),
                pl.BlockSpec((C, 1), lambda j, n, i: (0, 0)),
            ],
            out_specs=[
                pl.BlockSpec((1, C, 1), lambda j, n, i: (j, 0, 0)),
                pl.BlockSpec((1, C, 1), lambda j, n, i: (j, 0, 0)),
            ],
            scratch_shapes=[
                pltpu.VMEM((C, T), jnp.float32),
                pltpu.VMEM((C, T), jnp.float32),
            ],
        ),
        compiler_params=pltpu.CompilerParams(
            dimension_semantics=("parallel", "arbitrary", "arbitrary"),
            vmem_limit_bytes=vmem_limit),
    )(x3, g_bf, b_col)

    out3 = pl.pallas_call(
        functools.partial(_quant_kernel, qn=qn, qp=qp, inverse=inverse),
        out_shape=jax.ShapeDtypeStruct((N, C, HW), jnp.float32),
        grid_spec=pltpu.PrefetchScalarGridSpec(
            num_scalar_prefetch=0,
            grid=(N, HW // T),
            in_specs=[
                pl.BlockSpec((1, C, T), lambda n, i: (n, 0, i)),
                pl.BlockSpec((C, C), lambda n, i: (0, 0)),
                pl.BlockSpec((C, 1), lambda n, i: (0, 0)),
                pl.BlockSpec((C, 1), lambda n, i: (0, 0)),
                pl.BlockSpec((J, C, 1), lambda n, i: (0, 0, 0)),
                pl.BlockSpec((J, C, 1), lambda n, i: (0, 0, 0)),
            ],
            out_specs=pl.BlockSpec((1, C, T), lambda n, i: (n, 0, i)),
        ),
        compiler_params=pltpu.CompilerParams(
            dimension_semantics=("parallel", "arbitrary"),
            vmem_limit_bytes=vmem_limit),
    )(x3, g_bf, b_col, s1_col, mxp, mnp)

    return out3.reshape(N, C, H, W)


def kernel(x, gamma, beta, s1):
    return _gdn_forward(x, gamma, beta, s1, num=256, inverse=False)


# speedup vs baseline: 2.2955x; 2.2955x over previous
import functools

import jax
import jax.numpy as jnp
from jax import lax
from jax.experimental import pallas as pl
from jax.experimental.pallas import tpu as pltpu

SUBLANE = 8


def _stats_kernel(x_ref, gt_ref, b_ref, mx_ref, mn_ref, amx_ref, amn_ref,
                  *, tile_p):
    i = pl.program_id(1)

    @pl.when(i == 0)
    def _():
        amx_ref[...] = jnp.full(amx_ref.shape, -jnp.inf, amx_ref.dtype)
        amn_ref[...] = jnp.full(amn_ref.shape, jnp.inf, amn_ref.dtype)

    x = x_ref[...]
    x2 = (x * x).astype(jnp.bfloat16)
    xx = jnp.dot(x2, gt_ref[...], preferred_element_type=jnp.float32)

    c = xx.shape[-1]
    xx3 = xx.reshape(tile_p // SUBLANE, SUBLANE, c)
    amx_ref[...] = jnp.maximum(amx_ref[...], jnp.max(xx3, axis=0))
    amn_ref[...] = jnp.minimum(amn_ref[...], jnp.min(xx3, axis=0))

    @pl.when(i == pl.num_programs(1) - 1)
    def _():
        b = b_ref[...]
        mx_ref[0] = jnp.max(amx_ref[...], axis=0, keepdims=True) + b
        mn_ref[0] = jnp.min(amn_ref[...], axis=0, keepdims=True) + b


def _quant_kernel(x_ref, gt_ref, b_ref, s1_ref, mxp_ref, mnp_ref, o_ref,
                  *, qn, qp, inverse):
    mx = jnp.max(mxp_ref[...], axis=0)
    mn = jnp.min(mnp_ref[...], axis=0)
    qscl = (mx - mn) * (0.9 / (qp - qn))
    qoff = mn * 0.9 - qn * qscl
    inv_qscl = 1.0 / qscl

    x = x_ref[...]
    x2 = (x * x).astype(jnp.bfloat16)
    xx = jnp.dot(x2, gt_ref[...],
                 preferred_element_type=jnp.float32) + b_ref[...]

    x_hat = jnp.clip(jnp.round((xx - qoff) * inv_qscl), qn, qp)
    xq = x_hat * qscl + qoff

    if inverse:
        norm = jnp.sqrt(xq)
    else:
        norm = lax.rsqrt(xq)

    o_ref[...] = (s1_ref[...] * x) * norm


def _gdn_forward(x_nchw, gamma, beta, s1, *, num=256, inverse=False):
    N, C, H, W = x_nchw.shape
    P = N * H * W
    qn, qp = 0.0, float(num - 1)

    tile_p = 2048
    while P % (2 * tile_p) and tile_p > SUBLANE:
        tile_p //= 2
    steps = P // tile_p
    J = 2 if steps % 2 == 0 else 1

    x2d = jnp.transpose(x_nchw, (0, 2, 3, 1)).reshape(P, C)
    gt_bf = gamma.astype(jnp.bfloat16).T
    b_row = beta.astype(jnp.float32).reshape(1, C)
    s1_row = s1.astype(jnp.float32).reshape(1, C)

    vmem_limit = 48 * 1024 * 1024

    mxp, mnp = pl.pallas_call(
        functools.partial(_stats_kernel, tile_p=tile_p),
        out_shape=(
            jax.ShapeDtypeStruct((J, 1, C), jnp.float32),
            jax.ShapeDtypeStruct((J, 1, C), jnp.float32),
        ),
        grid_spec=pltpu.PrefetchScalarGridSpec(
            num_scalar_prefetch=0,
            grid=(J, steps // J),
            in_specs=[
                pl.BlockSpec((tile_p, C),
                             lambda j, i, _h=steps // J: (j * _h + i, 0)),
                pl.BlockSpec((C, C), lambda j, i: (0, 0)),
                pl.BlockSpec((1, C), lambda j, i: (0, 0)),
            ],
            out_specs=[
                pl.BlockSpec((1, 1, C), lambda j, i: (j, 0, 0)),
                pl.BlockSpec((1, 1, C), lambda j, i: (j, 0, 0)),
            ],
            scratch_shapes=[
                pltpu.VMEM((SUBLANE, C), jnp.float32),
                pltpu.VMEM((SUBLANE, C), jnp.float32),
            ],
        ),
        compiler_params=pltpu.CompilerParams(
            dimension_semantics=("parallel", "arbitrary"),
            vmem_limit_bytes=vmem_limit),
    )(x2d, gt_bf, b_row)

    out2d = pl.pallas_call(
        functools.partial(_quant_kernel, qn=qn, qp=qp, inverse=inverse),
        out_shape=jax.ShapeDtypeStruct((P, C), jnp.float32),
        grid_spec=pltpu.PrefetchScalarGridSpec(
            num_scalar_prefetch=0,
            grid=(steps,),
            in_specs=[
                pl.BlockSpec((tile_p, C), lambda i: (i, 0)),
                pl.BlockSpec((C, C), lambda i: (0, 0)),
                pl.BlockSpec((1, C), lambda i: (0, 0)),
                pl.BlockSpec((1, C), lambda i: (0, 0)),
                pl.BlockSpec((J, 1, C), lambda i: (0, 0, 0)),
                pl.BlockSpec((J, 1, C), lambda i: (0, 0, 0)),
            ],
            out_specs=pl.BlockSpec((tile_p, C), lambda i: (i, 0)),
        ),
        compiler_params=pltpu.CompilerParams(
            dimension_semantics=("parallel",),
            vmem_limit_bytes=vmem_limit),
    )(x2d, gt_bf, b_row, s1_row, mxp, mnp)

    return out2d.reshape(N, H, W, C).transpose(0, 3, 1, 2)


def kernel(x, gamma, beta, s1):
    return _gdn_forward(x, gamma, beta, s1, num=256, inverse=False)


# speedup vs baseline: 2.8555x; 1.2440x over previous
import functools

import jax
import jax.numpy as jnp
from jax import lax
from jax.experimental import pallas as pl
from jax.experimental.pallas import tpu as pltpu

SUBLANE = 8


def _stats_kernel(x_ref, gt_ref, b_ref, mx_ref, mn_ref, amx_ref, amn_ref,
                  *, tile_p):
    i = pl.program_id(1)

    @pl.when(i == 0)
    def _():
        amx_ref[...] = jnp.full(amx_ref.shape, -jnp.inf, amx_ref.dtype)
        amn_ref[...] = jnp.full(amn_ref.shape, jnp.inf, amn_ref.dtype)

    x = x_ref[...]
    x2 = (x * x).astype(jnp.bfloat16)
    xx = jnp.dot(x2, gt_ref[...], preferred_element_type=jnp.float32)

    c = xx.shape[-1]
    xx3 = xx.reshape(tile_p // SUBLANE, SUBLANE, c)
    amx_ref[...] = jnp.maximum(amx_ref[...], jnp.max(xx3, axis=0))
    amn_ref[...] = jnp.minimum(amn_ref[...], jnp.min(xx3, axis=0))

    @pl.when(i == pl.num_programs(1) - 1)
    def _():
        b = b_ref[...]
        mx_ref[0] = jnp.max(amx_ref[...], axis=0, keepdims=True) + b
        mn_ref[0] = jnp.min(amn_ref[...], axis=0, keepdims=True) + b


def _quant_kernel(x_ref, gt_ref, b_ref, s1_ref, mxp_ref, mnp_ref, o_ref,
                  *, qn, qp, inverse):
    mx = jnp.max(mxp_ref[...], axis=0)
    mn = jnp.min(mnp_ref[...], axis=0)
    qscl = (mx - mn) * (0.9 / (qp - qn))
    qoff = mn * 0.9 - qn * qscl
    inv_qscl = 1.0 / qscl

    x = x_ref[...]
    x2 = (x * x).astype(jnp.bfloat16)
    xx = jnp.dot(x2, gt_ref[...],
                 preferred_element_type=jnp.float32) + b_ref[...]

    x_hat = jnp.clip(jnp.round((xx - qoff) * inv_qscl), qn, qp)
    xq = x_hat * qscl + qoff

    if inverse:
        norm = jnp.sqrt(xq)
    else:
        norm = lax.rsqrt(xq)

    o_ref[...] = (s1_ref[...] * x) * norm


def _gdn_forward(x_nchw, gamma, beta, s1, *, num=256, inverse=False):
    N, C, H, W = x_nchw.shape
    P = N * H * W
    qn, qp = 0.0, float(num - 1)

    tile_p = 4096
    while P % (2 * tile_p) and tile_p > SUBLANE:
        tile_p //= 2
    steps = P // tile_p
    J = 2 if steps % 2 == 0 else 1

    x2d = jnp.transpose(x_nchw, (0, 2, 3, 1)).reshape(P, C)
    gt_bf = gamma.astype(jnp.bfloat16).T
    b_row = beta.astype(jnp.float32).reshape(1, C)
    s1_row = s1.astype(jnp.float32).reshape(1, C)

    vmem_limit = 48 * 1024 * 1024

    mxp, mnp = pl.pallas_call(
        functools.partial(_stats_kernel, tile_p=tile_p),
        out_shape=(
            jax.ShapeDtypeStruct((J, 1, C), jnp.float32),
            jax.ShapeDtypeStruct((J, 1, C), jnp.float32),
        ),
        grid_spec=pltpu.PrefetchScalarGridSpec(
            num_scalar_prefetch=0,
            grid=(J, steps // J),
            in_specs=[
                pl.BlockSpec((tile_p, C),
                             lambda j, i, _h=steps // J: (j * _h + i, 0)),
                pl.BlockSpec((C, C), lambda j, i: (0, 0)),
                pl.BlockSpec((1, C), lambda j, i: (0, 0)),
            ],
            out_specs=[
                pl.BlockSpec((1, 1, C), lambda j, i: (j, 0, 0)),
                pl.BlockSpec((1, 1, C), lambda j, i: (j, 0, 0)),
            ],
            scratch_shapes=[
                pltpu.VMEM((SUBLANE, C), jnp.float32),
                pltpu.VMEM((SUBLANE, C), jnp.float32),
            ],
        ),
        compiler_params=pltpu.CompilerParams(
            dimension_semantics=("parallel", "arbitrary"),
            vmem_limit_bytes=vmem_limit),
    )(x2d, gt_bf, b_row)

    out2d = pl.pallas_call(
        functools.partial(_quant_kernel, qn=qn, qp=qp, inverse=inverse),
        out_shape=jax.ShapeDtypeStruct((P, C), jnp.float32),
        grid_spec=pltpu.PrefetchScalarGridSpec(
            num_scalar_prefetch=0,
            grid=(steps,),
            in_specs=[
                pl.BlockSpec((tile_p, C), lambda i: (i, 0)),
                pl.BlockSpec((C, C), lambda i: (0, 0)),
                pl.BlockSpec((1, C), lambda i: (0, 0)),
                pl.BlockSpec((1, C), lambda i: (0, 0)),
                pl.BlockSpec((J, 1, C), lambda i: (0, 0, 0)),
                pl.BlockSpec((J, 1, C), lambda i: (0, 0, 0)),
            ],
            out_specs=pl.BlockSpec((tile_p, C), lambda i: (i, 0)),
        ),
        compiler_params=pltpu.CompilerParams(
            dimension_semantics=("parallel",),
            vmem_limit_bytes=vmem_limit),
    )(x2d, gt_bf, b_row, s1_row, mxp, mnp)

    return out2d.reshape(N, H, W, C).transpose(0, 3, 1, 2)


def kernel(x, gamma, beta, s1):
    return _gdn_forward(x, gamma, beta, s1, num=256, inverse=False)


# speedup vs baseline: 3.1039x; 1.0870x over previous
import functools

import jax
import jax.numpy as jnp
from jax import lax
from jax.experimental import pallas as pl
from jax.experimental.pallas import tpu as pltpu

SUBLANE = 8


def _stats_kernel(x_ref, gt_ref, b_ref, mx_ref, mn_ref, amx_ref, amn_ref,
                  *, tile_p):
    i = pl.program_id(1)

    @pl.when(i == 0)
    def _():
        amx_ref[...] = jnp.full(amx_ref.shape, -jnp.inf, amx_ref.dtype)
        amn_ref[...] = jnp.full(amn_ref.shape, jnp.inf, amn_ref.dtype)

    x = x_ref[...]
    x2 = (x * x).astype(jnp.bfloat16)
    xx = jnp.dot(x2, gt_ref[...], preferred_element_type=jnp.float32)

    c = xx.shape[-1]
    xx3 = xx.reshape(tile_p // SUBLANE, SUBLANE, c)
    amx_ref[...] = jnp.maximum(amx_ref[...], jnp.max(xx3, axis=0))
    amn_ref[...] = jnp.minimum(amn_ref[...], jnp.min(xx3, axis=0))

    @pl.when(i == pl.num_programs(1) - 1)
    def _():
        b = b_ref[...]
        mx_ref[0] = jnp.max(amx_ref[...], axis=0, keepdims=True) + b
        mn_ref[0] = jnp.min(amn_ref[...], axis=0, keepdims=True) + b


def _quant_kernel(x_ref, gt_ref, b_ref, s1_ref, mxp_ref, mnp_ref, o_ref,
                  *, qn, qp, inverse):
    mx = jnp.max(mxp_ref[...], axis=0)
    mn = jnp.min(mnp_ref[...], axis=0)
    qscl = (mx - mn) * (0.9 / (qp - qn))
    qoff = mn * 0.9 - qn * qscl
    inv_qscl = 1.0 / qscl

    x = x_ref[...]
    x2 = (x * x).astype(jnp.bfloat16)
    xx = jnp.dot(x2, gt_ref[...],
                 preferred_element_type=jnp.float32) + b_ref[...]

    x_hat = jnp.clip(jnp.round((xx - qoff) * inv_qscl), qn, qp)
    xq = x_hat * qscl + qoff

    if inverse:
        norm = jnp.sqrt(xq)
    else:
        norm = lax.rsqrt(xq)

    o_ref[...] = (s1_ref[...] * x) * norm


def _gdn_forward(x_nchw, gamma, beta, s1, *, num=256, inverse=False):
    N, C, H, W = x_nchw.shape
    P = N * H * W
    qn, qp = 0.0, float(num - 1)

    tile_p = 8192
    while P % (2 * tile_p) and tile_p > SUBLANE:
        tile_p //= 2
    steps = P // tile_p
    J = 2 if steps % 2 == 0 else 1

    x2d = jnp.transpose(x_nchw, (0, 2, 3, 1)).reshape(P, C)
    gt_bf = gamma.astype(jnp.bfloat16).T
    b_row = beta.astype(jnp.float32).reshape(1, C)
    s1_row = s1.astype(jnp.float32).reshape(1, C)

    vmem_limit = 48 * 1024 * 1024

    mxp, mnp = pl.pallas_call(
        functools.partial(_stats_kernel, tile_p=tile_p),
        out_shape=(
            jax.ShapeDtypeStruct((J, 1, C), jnp.float32),
            jax.ShapeDtypeStruct((J, 1, C), jnp.float32),
        ),
        grid_spec=pltpu.PrefetchScalarGridSpec(
            num_scalar_prefetch=0,
            grid=(J, steps // J),
            in_specs=[
                pl.BlockSpec((tile_p, C),
                             lambda j, i, _h=steps // J: (j * _h + i, 0)),
                pl.BlockSpec((C, C), lambda j, i: (0, 0)),
                pl.BlockSpec((1, C), lambda j, i: (0, 0)),
            ],
            out_specs=[
                pl.BlockSpec((1, 1, C), lambda j, i: (j, 0, 0)),
                pl.BlockSpec((1, 1, C), lambda j, i: (j, 0, 0)),
            ],
            scratch_shapes=[
                pltpu.VMEM((SUBLANE, C), jnp.float32),
                pltpu.VMEM((SUBLANE, C), jnp.float32),
            ],
        ),
        compiler_params=pltpu.CompilerParams(
            dimension_semantics=("parallel", "arbitrary"),
            vmem_limit_bytes=vmem_limit),
    )(x2d, gt_bf, b_row)

    out2d = pl.pallas_call(
        functools.partial(_quant_kernel, qn=qn, qp=qp, inverse=inverse),
        out_shape=jax.ShapeDtypeStruct((P, C), jnp.float32),
        grid_spec=pltpu.PrefetchScalarGridSpec(
            num_scalar_prefetch=0,
            grid=(steps,),
            in_specs=[
                pl.BlockSpec((tile_p, C), lambda i: (i, 0)),
                pl.BlockSpec((C, C), lambda i: (0, 0)),
                pl.BlockSpec((1, C), lambda i: (0, 0)),
                pl.BlockSpec((1, C), lambda i: (0, 0)),
                pl.BlockSpec((J, 1, C), lambda i: (0, 0, 0)),
                pl.BlockSpec((J, 1, C), lambda i: (0, 0, 0)),
            ],
            out_specs=pl.BlockSpec((tile_p, C), lambda i: (i, 0)),
        ),
        compiler_params=pltpu.CompilerParams(
            dimension_semantics=("parallel",),
            vmem_limit_bytes=vmem_limit),
    )(x2d, gt_bf, b_row, s1_row, mxp, mnp)

    return out2d.reshape(N, H, W, C).transpose(0, 3, 1, 2)


def kernel(x, gamma, beta, s1):
    return _gdn_forward(x, gamma, beta, s1, num=256, inverse=False)


# speedup vs baseline: 3.3489x; 1.0789x over previous
import functools

import jax
import jax.numpy as jnp
from jax import lax
from jax.experimental import pallas as pl
from jax.experimental.pallas import tpu as pltpu

SUBLANE = 8
RESIDENT = 8
ROTATING = 2


def _slot(b):
    return jnp.where(b < RESIDENT, b, RESIDENT + (b & (ROTATING - 1)))


def _fused_kernel(x_hbm, gt_ref, b_ref, s1_ref, o_ref,
                  xbuf, amx_ref, amn_ref, gst_ref, a_sems, b_sems,
                  *, n_steps, tile_p, qn, qp, inverse):
    t = pl.program_id(0)
    c = gt_ref.shape[-1]
    n_slots = RESIDENT + ROTATING

    def start_in(block, slot, sem):
        pltpu.make_async_copy(
            x_hbm.at[pl.ds(pl.multiple_of(block * tile_p, tile_p), tile_p), :],
            xbuf.at[slot], sem).start()

    @pl.when(t == 0)
    def _():
        amx_ref[...] = jnp.full(amx_ref.shape, -jnp.inf, amx_ref.dtype)
        amn_ref[...] = jnp.full(amn_ref.shape, jnp.inf, amn_ref.dtype)
        for b in range(min(n_slots, n_steps)):
            start_in(b, b, a_sems.at[b])

    @pl.when(t < n_steps)
    def _():
        pltpu.make_async_copy(xbuf.at[_slot(t)], xbuf.at[_slot(t)],
                              a_sems.at[t]).wait()
        x = xbuf[_slot(t)]
        x2 = (x * x).astype(jnp.bfloat16)
        xx = jnp.dot(x2, gt_ref[...], preferred_element_type=jnp.float32)
        xx3 = xx.reshape(tile_p // SUBLANE, SUBLANE, c)
        amx_ref[...] = jnp.maximum(amx_ref[...], jnp.max(xx3, axis=0))
        amn_ref[...] = jnp.minimum(amn_ref[...], jnp.min(xx3, axis=0))

        @pl.when((t >= RESIDENT) & (t + ROTATING < n_steps))
        def _():
            start_in(t + ROTATING, RESIDENT + (t & (ROTATING - 1)),
                     a_sems.at[t + ROTATING])

    @pl.when(t == n_steps)
    def _():
        b = b_ref[...]
        gst_ref[0:1, :] = jnp.max(amx_ref[...], axis=0, keepdims=True) + b
        gst_ref[1:2, :] = jnp.min(amn_ref[...], axis=0, keepdims=True) + b
        for r in range(ROTATING):
            if RESIDENT + r < n_steps:
                start_in(RESIDENT + r, RESIDENT + r,
                         b_sems.at[r])

    @pl.when(t >= n_steps)
    def _():
        s = t - n_steps

        @pl.when(s >= RESIDENT)
        def _():
            pltpu.make_async_copy(xbuf.at[_slot(s)], xbuf.at[_slot(s)],
                                  b_sems.at[s & (ROTATING - 1)]).wait()

        mx = gst_ref[0:1, :]
        mn = gst_ref[1:2, :]
        qscl = (mx - mn) * (0.9 / (qp - qn))
        qoff = mn * 0.9 - qn * qscl
        inv_qscl = 1.0 / qscl

        x = xbuf[_slot(s)]
        x2 = (x * x).astype(jnp.bfloat16)
        xx = jnp.dot(x2, gt_ref[...],
                     preferred_element_type=jnp.float32) + b_ref[...]

        x_hat = jnp.clip(jnp.round((xx - qoff) * inv_qscl), qn, qp)
        xq = x_hat * qscl + qoff

        if inverse:
            norm = jnp.sqrt(xq)
        else:
            norm = lax.rsqrt(xq)

        o_ref[...] = (s1_ref[...] * x) * norm

        @pl.when((s >= RESIDENT) & (s + ROTATING < n_steps))
        def _():
            start_in(s + ROTATING, RESIDENT + (s & (ROTATING - 1)),
                     b_sems.at[s & (ROTATING - 1)])


def _gdn_forward(x_nchw, gamma, beta, s1, *, num=256, inverse=False):
    N, C, H, W = x_nchw.shape
    P = N * H * W
    qn, qp = 0.0, float(num - 1)

    tile_p = 4096
    while P % tile_p and tile_p > SUBLANE:
        tile_p //= 2
    n_steps = P // tile_p

    x2d = jnp.transpose(x_nchw, (0, 2, 3, 1)).reshape(P, C)
    gt_bf = gamma.astype(jnp.bfloat16).T
    b_row = beta.astype(jnp.float32).reshape(1, C)
    s1_row = s1.astype(jnp.float32).reshape(1, C)

    out2d = pl.pallas_call(
        functools.partial(_fused_kernel, n_steps=n_steps, tile_p=tile_p,
                          qn=qn, qp=qp, inverse=inverse),
        out_shape=jax.ShapeDtypeStruct((P, C), jnp.float32),
        grid_spec=pltpu.PrefetchScalarGridSpec(
            num_scalar_prefetch=0,
            grid=(2 * n_steps,),
            in_specs=[
                pl.BlockSpec(memory_space=pl.ANY),
                pl.BlockSpec((C, C), lambda t: (0, 0)),
                pl.BlockSpec((1, C), lambda t: (0, 0)),
                pl.BlockSpec((1, C), lambda t: (0, 0)),
            ],
            out_specs=pl.BlockSpec(
                (tile_p, C),
                lambda t, _s=n_steps: (jnp.maximum(t - _s, 0), 0)),
            scratch_shapes=[
                pltpu.VMEM((RESIDENT + ROTATING, tile_p, C), jnp.float32),
                pltpu.VMEM((SUBLANE, C), jnp.float32),
                pltpu.VMEM((SUBLANE, C), jnp.float32),
                pltpu.VMEM((2, C), jnp.float32),
                pltpu.SemaphoreType.DMA((n_steps,)),
                pltpu.SemaphoreType.DMA((ROTATING,)),
            ],
        ),
        compiler_params=pltpu.CompilerParams(
            dimension_semantics=("arbitrary",),
            vmem_limit_bytes=54 * 1024 * 1024),
    )(x2d, gt_bf, b_row, s1_row)

    return out2d.reshape(N, H, W, C).transpose(0, 3, 1, 2)


def kernel(x, gamma, beta, s1):
    return _gdn_forward(x, gamma, beta, s1, num=256, inverse=False)


# speedup vs baseline: 3.5665x; 1.0650x over previous
import functools

import jax
import jax.numpy as jnp
from jax import lax
from jax.experimental import pallas as pl
from jax.experimental.pallas import tpu as pltpu

SUBLANE = 8
ROTATING = 2


def _fused_kernel(x_hbm, gt_ref, b_ref, s1_ref, o_ref,
                  xbf, xrot, amx_ref, amn_ref, gst_ref, a_sems,
                  *, n_steps, tile_p, qn, qp, inverse):
    t = pl.program_id(0)
    c = gt_ref.shape[-1]

    def start_in(block, slot, sem):
        pltpu.make_async_copy(
            x_hbm.at[pl.ds(pl.multiple_of(block * tile_p, tile_p), tile_p), :],
            xrot.at[slot], sem).start()

    @pl.when(t == 0)
    def _():
        amx_ref[...] = jnp.full(amx_ref.shape, -jnp.inf, amx_ref.dtype)
        amn_ref[...] = jnp.full(amn_ref.shape, jnp.inf, amn_ref.dtype)
        for b in range(min(ROTATING, n_steps)):
            start_in(b, b, a_sems.at[b])

    @pl.when(t < n_steps)
    def _():
        slot = t & (ROTATING - 1)
        pltpu.make_async_copy(xrot.at[slot], xrot.at[slot],
                              a_sems.at[t]).wait()
        x = xrot[slot]
        x2 = (x * x).astype(jnp.bfloat16)
        xx = jnp.dot(x2, gt_ref[...], preferred_element_type=jnp.float32)
        xx3 = xx.reshape(tile_p // SUBLANE, SUBLANE, c)
        amx_ref[...] = jnp.maximum(amx_ref[...], jnp.max(xx3, axis=0))
        amn_ref[...] = jnp.minimum(amn_ref[...], jnp.min(xx3, axis=0))
        xbf[t] = x.astype(jnp.bfloat16)

        @pl.when(t + ROTATING < n_steps)
        def _():
            start_in(t + ROTATING, slot, a_sems.at[t + ROTATING])

    @pl.when(t == n_steps)
    def _():
        b = b_ref[...]
        gst_ref[0:1, :] = jnp.max(amx_ref[...], axis=0, keepdims=True) + b
        gst_ref[1:2, :] = jnp.min(amn_ref[...], axis=0, keepdims=True) + b

    @pl.when(t >= n_steps)
    def _():
        s = t - n_steps
        mx = gst_ref[0:1, :]
        mn = gst_ref[1:2, :]
        qscl = (mx - mn) * (0.9 / (qp - qn))
        qoff = mn * 0.9 - qn * qscl
        inv_qscl = 1.0 / qscl

        xb = xbf[s]
        x2 = xb * xb
        xx = jnp.dot(x2, gt_ref[...],
                     preferred_element_type=jnp.float32) + b_ref[...]

        x_hat = jnp.clip(jnp.round((xx - qoff) * inv_qscl), qn, qp)
        xq = x_hat * qscl + qoff

        if inverse:
            norm = jnp.sqrt(xq)
        else:
            norm = lax.rsqrt(xq)

        o_ref[...] = (s1_ref[...] * norm) * xb.astype(jnp.float32)


def _gdn_forward(x_nchw, gamma, beta, s1, *, num=256, inverse=False):
    N, C, H, W = x_nchw.shape
    P = N * H * W
    qn, qp = 0.0, float(num - 1)

    tile_p = 4096
    while P % tile_p and tile_p > SUBLANE:
        tile_p //= 2
    n_steps = P // tile_p

    x2d = jnp.transpose(x_nchw, (0, 2, 3, 1)).reshape(P, C)
    gt_bf = gamma.astype(jnp.bfloat16).T
    b_row = beta.astype(jnp.float32).reshape(1, C)
    s1_row = s1.astype(jnp.float32).reshape(1, C)

    out2d = pl.pallas_call(
        functools.partial(_fused_kernel, n_steps=n_steps, tile_p=tile_p,
                          qn=qn, qp=qp, inverse=inverse),
        out_shape=jax.ShapeDtypeStruct((P, C), jnp.float32),
        grid_spec=pltpu.PrefetchScalarGridSpec(
            num_scalar_prefetch=0,
            grid=(2 * n_steps,),
            in_specs=[
                pl.BlockSpec(memory_space=pl.ANY),
                pl.BlockSpec((C, C), lambda t: (0, 0)),
                pl.BlockSpec((1, C), lambda t: (0, 0)),
                pl.BlockSpec((1, C), lambda t: (0, 0)),
            ],
            out_specs=pl.BlockSpec(
                (tile_p, C),
                lambda t, _s=n_steps: (jnp.maximum(t - _s, 0), 0)),
            scratch_shapes=[
                pltpu.VMEM((n_steps, tile_p, C), jnp.bfloat16),
                pltpu.VMEM((ROTATING, tile_p, C), jnp.float32),
                pltpu.VMEM((SUBLANE, C), jnp.float32),
                pltpu.VMEM((SUBLANE, C), jnp.float32),
                pltpu.VMEM((2, C), jnp.float32),
                pltpu.SemaphoreType.DMA((n_steps,)),
            ],
        ),
        compiler_params=pltpu.CompilerParams(
            dimension_semantics=("arbitrary",),
            vmem_limit_bytes=54 * 1024 * 1024),
    )(x2d, gt_bf, b_row, s1_row)

    return out2d.reshape(N, H, W, C).transpose(0, 3, 1, 2)


def kernel(x, gamma, beta, s1):
    return _gdn_forward(x, gamma, beta, s1, num=256, inverse=False)
